# baseline (speedup 1.0000x reference)
import jax
import jax.numpy as jnp
from jax import lax
from jax.experimental import pallas as pl
from jax.experimental.pallas import tpu as pltpu

NUM_CLASSES = 1000
CHUNK = 64
NBUF = 4


def _onehot_body(tgt_ref, out_ref, scratch_ref, sem_ref):
    b, s = tgt_ref.shape
    n_steps = b // CHUNK

    def _copy(i, buf):
        return pltpu.make_async_copy(
            scratch_ref.at[buf],
            out_ref.at[pl.ds(i * CHUNK, CHUNK)],
            sem_ref.at[buf],
        )

    def step(i, carry):
        buf = lax.rem(i, NBUF)

        @pl.when(i >= NBUF)
        def _():
            _copy(i - NBUF, buf).wait()

        tgt = tgt_ref[pl.ds(i * CHUNK, CHUNK), :]
        iota = lax.broadcasted_iota(jnp.int32, (CHUNK, s, NUM_CLASSES), 2)
        scratch_ref[buf] = (iota == tgt[:, :, None]).astype(jnp.float32)
        _copy(i, buf).start()
        return carry

    lax.fori_loop(0, n_steps, step, 0)
    for j in range(NBUF):
        i = n_steps - NBUF + j
        _copy(i, i % NBUF).wait()


def kernel(target):
    b, s = target.shape
    return pl.pallas_call(
        _onehot_body,
        in_specs=[pl.BlockSpec(memory_space=pltpu.MemorySpace.VMEM)],
        out_specs=pl.BlockSpec(memory_space=pltpu.MemorySpace.HBM),
        out_shape=jax.ShapeDtypeStruct((b, s, NUM_CLASSES), jnp.float32),
        scratch_shapes=[
            pltpu.VMEM((NBUF, CHUNK, s, NUM_CLASSES), jnp.float32),
            pltpu.SemaphoreType.DMA((NBUF,)),
        ],
    )(target)
